# baseline (speedup 1.0000x reference)
import functools

import jax
import jax.numpy as jnp
from jax import lax
from jax.experimental import pallas as pl
from jax.experimental.pallas import tpu as pltpu
from jax.experimental.pallas import tpu_sc as plsc

_SEQ = 200
_HID = 64
_BATCH = 1024
_VPR = _HID // 16
_NW = 32
_IPW = _BATCH // _NW
_S0 = 104
_S1 = _SEQ - _S0
_NGRP = 13
_NIDX = _IPW * _SEQ
_NIDX_PAD = _NIDX + 208


def _emb_body(x_hbm, table_hbm, pos_hbm, out_hbm,
              idx_v, rows0, rows1, pe_v, gs0, gs1, ss0, ss1):
    wid = lax.axis_index("s") * 2 + lax.axis_index("c")
    base_item = wid * _IPW
    pltpu.sync_copy(pos_hbm, pe_v)
    pltpu.sync_copy(x_hbm.at[pl.ds(base_item * _SEQ, _NIDX)],
                    idx_v.at[pl.ds(0, _NIDX)])
    for i in range(_NIDX, _NIDX_PAD, 16):
        idx_v[pl.ds(i, 16)] = jnp.zeros((16,), jnp.int32)

    rows = (rows0, rows1)
    gsem = (gs0, gs1)
    ssem = (ss0, ss1)
    lane = jnp.arange(16, dtype=jnp.int32)
    zeros16 = jnp.zeros((16,), jnp.float32)

    def gathers(k, b):
        off = k * _SEQ
        c0 = pltpu.async_copy(
            table_hbm.at[idx_v.at[pl.ds(off, _S0)]],
            rows[b].at[pl.ds(0, _S0)], gsem[b])
        c1 = pltpu.async_copy(
            table_hbm.at[idx_v.at[pl.ds(off + _S0, _S1)]],
            rows[b].at[pl.ds(_S0, _S1)], gsem[b])
        return c0, c1

    def wait_gathers(b):
        pltpu.make_async_copy(
            table_hbm.at[idx_v.at[pl.ds(0, _S0)]],
            rows[b].at[pl.ds(0, _S0)], gsem[b]).wait()
        pltpu.make_async_copy(
            table_hbm.at[idx_v.at[pl.ds(0, _S1)]],
            rows[b].at[pl.ds(_S0, _S1)], gsem[b]).wait()

    def store(k, b):
        pltpu.async_copy(rows[b].at[pl.ds(0, _SEQ)],
                         out_hbm.at[base_item + k], ssem[b])

    def wait_store(b):
        pltpu.make_async_copy(rows[b].at[pl.ds(0, _SEQ)],
                              out_hbm.at[base_item], ssem[b]).wait()

    store(0, 0)
    store(1, 1)
    gathers(0, 0)

    def item_body(ko, carry):
        for b2 in range(2):
            k = ko * 2 + b2
            b = b2
            nb = 1 - b2
            wait_store(nb)
            gathers(k + 1, nb)
            wait_gathers(b)

            def grp_body(g, c2):
                iv = idx_v[pl.ds(k * _SEQ + g * 16, 16)]
                m = iv == 1

                def fixup():
                    rr = g * 16 + lane
                    for c in range(_HID):
                        plsc.store_scatter(
                            rows[b], [rr, jnp.zeros((16,), jnp.int32) + c],
                            zeros16, mask=m)

                lax.cond(jnp.any(m), fixup, lambda: None)
                return c2

            lax.fori_loop(0, _NGRP, grp_body, 0, unroll=False)

            def row_body(r, c2):
                for c in range(_VPR):
                    sl = pl.ds(c * 16, 16)
                    rows[b][r, sl] = rows[b][r, sl] + pe_v[r, sl]
                return c2

            lax.fori_loop(0, _SEQ, row_body, 0, unroll=False)
            store(k, b)
        return carry

    lax.fori_loop(0, _IPW // 2, item_body, 0, unroll=False)
    wait_gathers(0)
    wait_store(0)
    wait_store(1)


@functools.partial(
    pl.kernel,
    mesh=plsc.VectorSubcoreMesh(core_axis_name="c", subcore_axis_name="s"),
    compiler_params=pltpu.CompilerParams(
        needs_layout_passes=False, use_tc_tiling_on_sc=False),
    out_type=jax.ShapeDtypeStruct((_BATCH, _SEQ, _HID), jnp.float32),
    scratch_types=[
        pltpu.VMEM((_NIDX_PAD,), jnp.int32),
        pltpu.VMEM((_NGRP * 16, _HID), jnp.float32),
        pltpu.VMEM((_NGRP * 16, _HID), jnp.float32),
        pltpu.VMEM((_SEQ, _HID), jnp.float32),
        pltpu.SemaphoreType.DMA,
        pltpu.SemaphoreType.DMA,
        pltpu.SemaphoreType.DMA,
        pltpu.SemaphoreType.DMA,
    ],
)
def _emb_call(x_hbm, table_hbm, pos_hbm, out_hbm,
              idx_v, rows0, rows1, pe_v, gs0, gs1, ss0, ss1):
    _emb_body(x_hbm, table_hbm, pos_hbm, out_hbm,
              idx_v, rows0, rows1, pe_v, gs0, gs1, ss0, ss1)


def kernel(x, table, pos_emb):
    return _emb_call(x.astype(jnp.int32).reshape(-1), table, pos_emb)
